# baseline (speedup 1.0000x reference)
import functools

import jax
import jax.numpy as jnp
from jax import lax
from jax.experimental import pallas as pl
from jax.experimental.pallas import tpu as pltpu
from jax.experimental.pallas import tpu_sc as plsc

VOCAB = 1000000
DIM = 64
MAX_LEN = 200
BATCH = 4096

NUM_CORES = 2
NUM_SUBCORES = 16
NW = NUM_CORES * NUM_SUBCORES
ROWS = BATCH * MAX_LEN
PER_W = ROWS // NW
CHUNK = MAX_LEN
NCHUNK = PER_W // CHUNK
LANES = 16
CGROUPS = DIM // LANES


def _build_kernel():
    mesh = plsc.VectorSubcoreMesh(core_axis_name="c", subcore_axis_name="s")

    @functools.partial(
        pl.kernel,
        mesh=mesh,
        out_type=jax.ShapeDtypeStruct((ROWS, DIM), jnp.float32),
        scratch_types=[
            pltpu.VMEM((CHUNK,), jnp.int32),
            pltpu.VMEM((CHUNK, DIM), jnp.float32),
            pltpu.VMEM((MAX_LEN, DIM), jnp.float32),
            pltpu.SemaphoreType.DMA,
        ],
        compiler_params=pltpu.CompilerParams(use_tc_tiling_on_sc=False),
    )
    def gather_add(tok_hbm, table_hbm, pos_hbm, out_hbm, idx_v, rows_v, pos_v, sem):
        wid = lax.axis_index("s") * NUM_CORES + lax.axis_index("c")
        base = wid * PER_W
        pltpu.sync_copy(pos_hbm, pos_v)

        def chunk_body(ci, carry):
            start = base + ci * CHUNK
            pltpu.sync_copy(tok_hbm.at[pl.ds(start, CHUNK)], idx_v)
            pltpu.async_copy(table_hbm.at[idx_v], rows_v, sem).wait()

            def row_body(r, c2):
                for c in range(CGROUPS):
                    s = pl.ds(c * LANES, LANES)
                    rows_v[r, s] = rows_v[r, s] + pos_v[r, s]
                return c2

            lax.fori_loop(0, CHUNK, row_body, 0, unroll=2)
            pltpu.sync_copy(rows_v, out_hbm.at[pl.ds(start, CHUNK)])
            return carry

        lax.fori_loop(0, NCHUNK, chunk_body, 0)

    return gather_add


_GATHER_ADD = _build_kernel()


def kernel(tokens, embedding, pos_embedding):
    flat = tokens.reshape(ROWS).astype(jnp.int32)
    out = _GATHER_ADD(flat, embedding, pos_embedding)
    return out.reshape(BATCH, MAX_LEN, DIM)


# speedup vs baseline: 1.1125x; 1.1125x over previous
import functools

import jax
import jax.numpy as jnp
from jax import lax
from jax.experimental import pallas as pl
from jax.experimental.pallas import tpu as pltpu
from jax.experimental.pallas import tpu_sc as plsc

VOCAB = 1000000
DIM = 64
MAX_LEN = 200
BATCH = 4096

NUM_CORES = 2
NUM_SUBCORES = 16
NW = NUM_CORES * NUM_SUBCORES
ROWS = BATCH * MAX_LEN
PER_W = ROWS // NW
CHUNK = 2 * MAX_LEN
NCHUNK = PER_W // CHUNK
LANES = 16
CGROUPS = DIM // LANES


def _build_kernel():
    mesh = plsc.VectorSubcoreMesh(core_axis_name="c", subcore_axis_name="s")

    @functools.partial(
        pl.kernel,
        mesh=mesh,
        out_type=jax.ShapeDtypeStruct((ROWS, DIM), jnp.float32),
        scratch_types=[
            pltpu.VMEM((2, CHUNK), jnp.int32),
            pltpu.VMEM((CHUNK, DIM), jnp.float32),
            pltpu.VMEM((CHUNK, DIM), jnp.float32),
            pltpu.VMEM((CHUNK, DIM), jnp.float32),
            pltpu.SemaphoreType.DMA,
            pltpu.SemaphoreType.DMA,
        ],
        compiler_params=pltpu.CompilerParams(use_tc_tiling_on_sc=False),
    )
    def gather_add(tok_hbm, table_hbm, pos_hbm, out_hbm,
                   idx_v, rows0_v, rows1_v, pos_v, sem0, sem1):
        wid = lax.axis_index("s") * NUM_CORES + lax.axis_index("c")
        base = wid * PER_W
        rows_bufs = (rows0_v, rows1_v)
        sems = (sem0, sem1)

        for rep in range(CHUNK // MAX_LEN):
            pltpu.sync_copy(pos_hbm, pos_v.at[pl.ds(rep * MAX_LEN, MAX_LEN)])

        pltpu.sync_copy(tok_hbm.at[pl.ds(base, CHUNK)], idx_v.at[0])
        pltpu.async_copy(table_hbm.at[idx_v.at[0]], rows0_v, sem0)

        def pair_body(g, carry):
            ci0 = 2 * g
            for b in range(2):
                ci = ci0 + b
                rows_v = rows_bufs[b]
                nxt_rows = rows_bufs[1 - b]

                @pl.when(ci + 1 < NCHUNK)
                def _():
                    nstart = base + (ci + 1) * CHUNK
                    pltpu.sync_copy(tok_hbm.at[pl.ds(nstart, CHUNK)],
                                    idx_v.at[1 - b])
                    pltpu.async_copy(table_hbm.at[idx_v.at[1 - b]],
                                     nxt_rows, sems[1 - b])

                pltpu.make_async_copy(table_hbm.at[idx_v.at[b]],
                                      rows_v, sems[b]).wait()

                def row_body(r, c2):
                    for c in range(CGROUPS):
                        s = pl.ds(c * LANES, LANES)
                        rows_v[r, s] = rows_v[r, s] + pos_v[r, s]
                    return c2

                lax.fori_loop(0, CHUNK, row_body, 0, unroll=4)
                pltpu.sync_copy(rows_v,
                                out_hbm.at[pl.ds(base + ci * CHUNK, CHUNK)])
            return carry

        lax.fori_loop(0, NCHUNK // 2, pair_body, 0)

    return gather_add


_GATHER_ADD = _build_kernel()


def kernel(tokens, embedding, pos_embedding):
    flat = tokens.reshape(ROWS).astype(jnp.int32)
    out = _GATHER_ADD(flat, embedding, pos_embedding)
    return out.reshape(BATCH, MAX_LEN, DIM)
